# Initial kernel scaffold; baseline (speedup 1.0000x reference)
import jax
import jax.numpy as jnp
from jax.experimental import pallas as pl


def kernel(x, edge_index, batch, le_lin1_W, le_lin1_b, le_lin2_W, le_lin3_W, le_lin3_b, cg_out_W, cg_out_b, cg_root_W, fc1_W, fc1_b, fc2_W, fc2_b):
    raise NotImplementedError("write your pallas kernel here")


# speedup vs baseline: 20.9849x; 20.9849x over previous
import functools
import jax
import jax.numpy as jnp
from jax import lax
from jax.experimental import pallas as pl
from jax.experimental.pallas import tpu as pltpu
from jax.experimental.pallas import tpu_sc as plsc

N = 10000
E = 320000
D = 128
G = 128

NC = 2
NS = 16
NW = NC * NS
K = 80
EPW = E // NW
NCHUNK = EPW // K
NPAD = 10240
RPT = NPAD // NS


def _make_sc_agg(with_deg):
    mesh = plsc.VectorSubcoreMesh(core_axis_name="c", subcore_axis_name="s")
    out_type = [jax.ShapeDtypeStruct((NC, NPAD, D), jnp.float32)]
    if with_deg:
        out_type.append(jax.ShapeDtypeStruct((NW, N), jnp.int32))
    scratch = [
        pltpu.VMEM((EPW,), jnp.int32),
        pltpu.VMEM((K,), jnp.int32),
        pltpu.VMEM((K,), jnp.int32),
        pltpu.VMEM((K, D), jnp.float32),
        pltpu.VMEM((K, D), jnp.float32),
        pltpu.VMEM_SHARED((NPAD, D), jnp.float32),
        pltpu.SemaphoreType.DMA,
        pltpu.SemaphoreType.DMA,
        pltpu.SemaphoreType.DMA,
        pltpu.SemaphoreType.DMA,
    ]
    if with_deg:
        scratch.append(pltpu.VMEM((N,), jnp.int32))

    @functools.partial(pl.kernel, mesh=mesh, out_type=out_type,
                       scratch_types=scratch,
                       compiler_params=pltpu.CompilerParams(
                           needs_layout_passes=False))
    def agg(x_hbm, edges_hbm, *refs):
        if with_deg:
            (out_hbm, deg_hbm, srcall, dv0, dv1, rows0, rows1, acc_sh,
             gs0, gs1, dsm0, dsm1, comb_v) = refs
        else:
            (out_hbm, srcall, dv0, dv1, rows0, rows1, acc_sh,
             gs0, gs1, dsm0, dsm1) = refs
        dv = (dv0, dv1)
        rows = (rows0, rows1)
        gsem = (gs0, gs1)
        dsem = (dsm0, dsm1)
        c = lax.axis_index("c")
        s = lax.axis_index("s")
        wid = c * NS + s
        ebase = wid * EPW

        z16 = jnp.zeros((16,), jnp.float32)
        zi16 = jnp.zeros((16,), jnp.int32)

        def zero_rows(i, _):
            r = i // (D // 16)
            col = (i % (D // 16)) * 16
            rows0[r, pl.ds(col, 16)] = z16
            return _
        lax.fori_loop(0, K * (D // 16), zero_rows, 0)

        row0 = s * RPT
        for j in range(RPT // K):
            pltpu.sync_copy(rows0, acc_sh.at[pl.ds(row0 + j * K, K)])

        if with_deg:
            def zero_deg(i, _):
                comb_v[pl.ds(i * 16, 16)] = zi16
                return _
            lax.fori_loop(0, N // 16, zero_deg, 0)

        plsc.subcore_barrier()

        pltpu.sync_copy(edges_hbm.at[pl.ds(ebase, EPW)], srcall)
        pltpu.async_copy(edges_hbm.at[pl.ds(E + ebase, K)], dv0, dsm0)
        pltpu.async_copy(x_hbm.at[srcall.at[pl.ds(0, K)]], rows0, gs0)

        def do_chunk(i, b):
            @pl.when(i + 1 < NCHUNK)
            def _():
                nb = 1 - b
                pltpu.async_copy(
                    edges_hbm.at[pl.ds(E + ebase + (i + 1) * K, K)],
                    dv[nb], dsem[nb])
                pltpu.async_copy(
                    x_hbm.at[srcall.at[pl.ds((i + 1) * K, K)]],
                    rows[nb], gsem[nb])
            pltpu.make_async_copy(
                edges_hbm.at[pl.ds(0, K)], dv[b], dsem[b]).wait()
            if with_deg:
                for j in range(K // 16):
                    d16 = dv[b][pl.ds(j * 16, 16)]
                    s16 = srcall[pl.ds(i * K + j * 16, 16)]
                    vals = jnp.where(s16 == d16, 16385, 16384)
                    plsc.addupdate_scatter(comb_v, [d16], vals)
            pltpu.make_async_copy(
                x_hbm.at[pl.ds(0, K)], rows[b], gsem[b]).wait()
            pltpu.sync_copy(rows[b], acc_sh.at[dv[b]], add=True)

        def pair(g, carry):
            do_chunk(2 * g, 0)

            @pl.when(2 * g + 1 < NCHUNK)
            def _():
                do_chunk(2 * g + 1, 1)
            return carry
        lax.fori_loop(0, (NCHUNK + 1) // 2, pair, 0)

        plsc.subcore_barrier()

        pltpu.sync_copy(acc_sh.at[pl.ds(row0, RPT)],
                        out_hbm.at[c, pl.ds(row0, RPT)])
        if with_deg:
            pltpu.sync_copy(comb_v, deg_hbm.at[wid])

    return agg


_sc_agg_deg = _make_sc_agg(True)
_sc_agg_plain = _make_sc_agg(False)

BA = 2000


def _tc_h1_body(x_r, agg_r, degp_r, W1_r, b1_r, W2_r, W3_r, b3_r, h1_r):
    x = x_r[...]
    onesw = jnp.ones((NW, 1), jnp.float32)
    idegp = (degp_r[0] >> 14).astype(jnp.float32)
    indeg = jnp.dot(idegp, onesw,
                    preferred_element_type=jnp.float32)
    a = jnp.dot(x, W1_r[...], preferred_element_type=jnp.float32) + b1_r[...]
    aggx = agg_r[0] + agg_r[1]
    h = (indeg * a
         - jnp.dot(aggx, W2_r[...], preferred_element_type=jnp.float32)
         + jnp.dot(x, W3_r[...], preferred_element_type=jnp.float32)
         + b3_r[...])
    h1_r[...] = jnp.maximum(h, 0.0)


def _tc_h1(x, agg, degp, W1, b1, W2, W3, b3):
    full = lambda i: (0, 0)
    blk = lambda i: (i, 0)
    return pl.pallas_call(
        _tc_h1_body,
        grid=(N // BA,),
        in_specs=[
            pl.BlockSpec((BA, D), blk),
            pl.BlockSpec((NC, BA, D), lambda i: (0, i, 0)),
            pl.BlockSpec((1, BA, NW), lambda i: (i, 0, 0)),
            pl.BlockSpec((D, D), full),
            pl.BlockSpec((1, D), full),
            pl.BlockSpec((D, D), full),
            pl.BlockSpec((D, D), full),
            pl.BlockSpec((1, D), full),
        ],
        out_specs=pl.BlockSpec((BA, D), blk),
        out_shape=jax.ShapeDtypeStruct((N, D), jnp.float32),
    )(x, agg, degp, W1, b1, W2, W3, b3)


def _tc_h3_body(h1_r, agg_r, degp_r, batch_r,
                co_W_r, co_b_r, cr_W_r, f1_W_r, f1_b_r, f2_W_r, f2_b_r,
                out_r, psum, cnt):
    i = pl.program_id(0)
    nb = pl.num_programs(0)
    h1 = h1_r[...]
    onesw = jnp.ones((NW, 1), jnp.float32)
    dg = (((0,), (0,)), ((), ()))
    comb = degp_r[0]
    idegp = (comb >> 14).astype(jnp.float32)
    sdegp = (comb & 16383).astype(jnp.float32)
    indeg = jnp.dot(idegp, onesw, preferred_element_type=jnp.float32)
    selfc = jnp.dot(sdegp, onesw, preferred_element_type=jnp.float32)
    deg = jnp.maximum(indeg - selfc + 1.0, 1.0)
    dinv = 1.0 / deg
    aggh = agg_r[0] + agg_r[1] - selfc * h1
    agg2 = dinv * (aggh + h1)
    h2 = (jnp.dot(agg2, co_W_r[...], preferred_element_type=jnp.float32)
          + co_b_r[...]
          + jnp.dot(h1, cr_W_r[...], preferred_element_type=jnp.float32))
    h3 = jnp.maximum(h2, 0.0)

    seg = lax.broadcasted_iota(jnp.int32, (1, G), 1)
    onehot = (batch_r[...] == seg).astype(jnp.float32)
    pp = lax.dot_general(onehot, h3, dg,
                         preferred_element_type=jnp.float32)
    ones_b = jnp.ones((BA, 1), jnp.float32)
    cc = lax.dot_general(onehot, ones_b, dg,
                         preferred_element_type=jnp.float32)

    @pl.when(i == 0)
    def _():
        psum[...] = pp
        cnt[...] = cc

    @pl.when(i > 0)
    def _():
        psum[...] += pp
        cnt[...] += cc

    @pl.when(i == nb - 1)
    def _():
        pooled = psum[...] / jnp.maximum(cnt[...], 1.0)
        h4 = jnp.maximum(
            jnp.dot(pooled, f1_W_r[...], preferred_element_type=jnp.float32)
            + f1_b_r[...], 0.0)
        out_r[...] = (jnp.dot(h4, f2_W_r[...],
                              preferred_element_type=jnp.float32)
                      + f2_b_r[...])


def _tc_h3(h1, agg, degp, batch2, co_W, co_b, cr_W,
           f1_W, f1_b, f2_Wp, f2_bp):
    full = lambda i: (0, 0)
    blk = lambda i: (i, 0)
    return pl.pallas_call(
        _tc_h3_body,
        grid=(N // BA,),
        in_specs=[
            pl.BlockSpec((BA, D), blk),
            pl.BlockSpec((NC, BA, D), lambda i: (0, i, 0)),
            pl.BlockSpec((1, BA, NW), lambda i: (i, 0, 0)),
            pl.BlockSpec((BA, 1), blk),
            pl.BlockSpec((D, D), full),
            pl.BlockSpec((1, D), full),
            pl.BlockSpec((D, D), full),
            pl.BlockSpec((D, D), full),
            pl.BlockSpec((1, D), full),
            pl.BlockSpec((D, G), full),
            pl.BlockSpec((1, G), full),
        ],
        out_specs=pl.BlockSpec((G, G), full),
        out_shape=jax.ShapeDtypeStruct((G, G), jnp.float32),
        scratch_shapes=[
            pltpu.VMEM((G, D), jnp.float32),
            pltpu.VMEM((G, 1), jnp.float32),
        ],
    )(h1, agg, degp, batch2, co_W, co_b, cr_W, f1_W, f1_b,
      f2_Wp, f2_bp)


def kernel(x, edge_index, batch, le_lin1_W, le_lin1_b, le_lin2_W, le_lin3_W,
           le_lin3_b, cg_out_W, cg_out_b, cg_root_W, fc1_W, fc1_b, fc2_W,
           fc2_b):
    C = fc2_W.shape[1]
    edges_flat = edge_index.reshape(-1)
    aggx, degp = _sc_agg_deg(x, edges_flat)
    NB = N // BA
    degp3 = degp.reshape(NW, NB, BA).transpose(1, 2, 0)
    h1 = _tc_h1(x, aggx, degp3,
                le_lin1_W, le_lin1_b[None, :], le_lin2_W, le_lin3_W,
                le_lin3_b[None, :])
    aggh = _sc_agg_plain(h1, edges_flat)
    if isinstance(aggh, (list, tuple)):
        aggh = aggh[0]
    f2_Wp = jnp.zeros((D, G), jnp.float32).at[:, :C].set(fc2_W)
    f2_bp = jnp.zeros((1, G), jnp.float32).at[:, :C].set(fc2_b)
    out = _tc_h3(h1, aggh, degp3, batch[:, None],
                 cg_out_W, cg_out_b[None, :], cg_root_W,
                 fc1_W, fc1_b[None, :], f2_Wp, f2_bp)
    return out[:, :C]
